# baseline (speedup 1.0000x reference)
import functools

import jax
import jax.numpy as jnp
from jax.experimental import pallas as pl
from jax.experimental.pallas import tpu as pltpu
from jax.experimental.pallas import tpu_sc as plsc

_NUM_HEADS = 16
_NUM_KV_HEADS = 4
_QPK = _NUM_HEADS // _NUM_KV_HEADS
_HEAD = 128
_SCALE = 0.08838834764831845
_PAGE = 16
_NUM_PAGES = 128
_L = _PAGE * _NUM_PAGES
_BATCH = 16
_NEG = float(jnp.finfo(jnp.float32).min)


def _attn_body(ctx_ref, sm_ref, sp_ref, q_ref, k_ref, v_ref, kn_ref, vn_ref,
               o_ref):
    b = pl.program_id(0)
    ctx = ctx_ref[b]

    q = q_ref[0, 0]
    k = k_ref[0, 0]
    v = v_ref[0, 0]
    sp = sp_ref[0]

    ov = jnp.zeros((1, _L), jnp.int32)
    for bp in range(_BATCH):
        ov = jnp.where(sp == sm_ref[bp], bp + 1, ov)
    onehot = (jax.lax.broadcasted_iota(jnp.int32, (_BATCH, _L), 0) + 1 == ov
              ).astype(jnp.float32)
    kov = jax.lax.dot(kn_ref[0], onehot,
                      preferred_element_type=jnp.float32)
    vov = jax.lax.dot(vn_ref[0], onehot,
                      preferred_element_type=jnp.float32)
    has = ov > 0
    k = jnp.where(has, kov, k)
    v = jnp.where(has, vov, v)

    s = jax.lax.dot_general(q, k, (((1,), (0,)), ((), ())),
                            preferred_element_type=jnp.float32) * _SCALE
    mask = jax.lax.broadcasted_iota(jnp.int32, (1, _L), 1) < ctx
    s = jnp.where(mask, s, _NEG)
    m = jnp.max(s, axis=1, keepdims=True)
    e = jnp.exp(s - m)
    den = jnp.sum(e, axis=1, keepdims=True)
    o = jax.lax.dot_general(e, v, (((1,), (1,)), ((), ())),
                            preferred_element_type=jnp.float32)
    o_ref[0, 0] = o / den


def _attention(kg, vg, q_r, knT, vnT, slot_pos, slot_map, context_lens,
               interpret=False):
    grid = (_BATCH, _NUM_KV_HEADS)
    return pl.pallas_call(
        _attn_body,
        grid=grid,
        in_specs=[
            pl.BlockSpec(memory_space=pltpu.SMEM),
            pl.BlockSpec(memory_space=pltpu.SMEM),
            pl.BlockSpec((1, 1, _L), lambda b, h: (b, 0, 0)),
            pl.BlockSpec((1, 1, _QPK, _HEAD), lambda b, h: (b, h, 0, 0)),
            pl.BlockSpec((1, 1, _HEAD, _L), lambda b, h: (b, h, 0, 0)),
            pl.BlockSpec((1, 1, _HEAD, _L), lambda b, h: (b, h, 0, 0)),
            pl.BlockSpec((1, _HEAD, _BATCH), lambda b, h: (h, 0, 0)),
            pl.BlockSpec((1, _HEAD, _BATCH), lambda b, h: (h, 0, 0)),
        ],
        out_specs=pl.BlockSpec((1, 1, _QPK, _HEAD), lambda b, h: (b, h, 0, 0)),
        out_shape=jax.ShapeDtypeStruct(
            (_BATCH, _NUM_KV_HEADS, _QPK, _HEAD), jnp.float32),
        interpret=interpret,
    )(context_lens, slot_map, slot_pos, q_r, kg, vg, knT, vnT)


_HALF_PAGES = _NUM_PAGES // 2
_CHUNK = 8


def _gather_body(kc, vc, bt, kg, vg, idx_v, buf, gsem, wsem):
    wid = jax.lax.axis_index("s") * 2 + jax.lax.axis_index("c")
    b = wid // 2
    half = wid % 2
    pltpu.sync_copy(bt.at[pl.ds(b * _NUM_PAGES + half * _HALF_PAGES,
                                _HALF_PAGES)], idx_v)

    def chunk(c, _):
        page0 = half * _HALF_PAGES + c * _CHUNK
        idx = idx_v.at[pl.ds(c * _CHUNK, _CHUNK)]
        for src, dst in ((kc, kg), (vc, vg)):
            pltpu.async_copy(src.at[idx], buf, gsem).wait()
            cps = []
            for p in range(_CHUNK):
                for h in range(_NUM_KV_HEADS):
                    cps.append(pltpu.make_async_copy(
                        buf.at[p, h], dst.at[b, h, :, page0 + p, :], wsem))
            for cp in cps:
                cp.start()
            for cp in cps:
                cp.wait()
        return 0

    jax.lax.fori_loop(0, _HALF_PAGES // _CHUNK, chunk, 0)


def _sc_gather(key_cache, value_cache, bt_flat):
    out = jax.ShapeDtypeStruct(
        (_BATCH, _NUM_KV_HEADS, _HEAD, _NUM_PAGES, _PAGE), jnp.float32)
    fn = pl.kernel(
        _gather_body,
        out_type=(out, out),
        mesh=plsc.VectorSubcoreMesh(core_axis_name="c", subcore_axis_name="s"),
        compiler_params=pltpu.CompilerParams(use_tc_tiling_on_sc=False),
        scratch_types=[
            pltpu.VMEM((_HALF_PAGES,), jnp.int32),
            pltpu.VMEM((_CHUNK, _NUM_KV_HEADS, _HEAD, _PAGE), jnp.float32),
            pltpu.SemaphoreType.DMA,
            pltpu.SemaphoreType.DMA,
        ],
    )
    return fn(key_cache, value_cache, bt_flat)


def kernel(query, key, value, key_cache, value_cache, slot_mapping,
           block_tables, context_lens):
    batch, q_len, hidden = query.shape
    q_r = query.reshape(_BATCH, _NUM_KV_HEADS, _QPK, _HEAD)
    knT = jnp.transpose(key.reshape(_BATCH, _NUM_KV_HEADS, _HEAD), (1, 2, 0))
    vnT = jnp.transpose(value.reshape(_BATCH, _NUM_KV_HEADS, _HEAD), (1, 2, 0))
    sm_flat = slot_mapping.reshape(-1).astype(jnp.int32)
    slot_pos = (block_tables[:, :, None] * _PAGE +
                jnp.arange(_PAGE, dtype=jnp.int32)[None, None, :]
                ).reshape(_BATCH, 1, _L)
    bt_flat = block_tables.reshape(-1).astype(jnp.int32)

    kg, vg = _sc_gather(key_cache, value_cache, bt_flat)
    kg = kg.reshape(_BATCH, _NUM_KV_HEADS, _HEAD, _L)
    vg = vg.reshape(_BATCH, _NUM_KV_HEADS, _HEAD, _L)
    out = _attention(kg, vg, q_r, knT, vnT, slot_pos, sm_flat,
                     context_lens.astype(jnp.int32))
    return out.reshape(batch, q_len, hidden)


# speedup vs baseline: 1.4516x; 1.4516x over previous
import functools

import jax
import jax.numpy as jnp
from jax.experimental import pallas as pl
from jax.experimental.pallas import tpu as pltpu
from jax.experimental.pallas import tpu_sc as plsc

_NUM_HEADS = 16
_NUM_KV_HEADS = 4
_QPK = _NUM_HEADS // _NUM_KV_HEADS
_HEAD = 128
_SCALE = 0.08838834764831845
_PAGE = 16
_NUM_PAGES = 128
_L = _PAGE * _NUM_PAGES
_BATCH = 16
_NEG = float(jnp.finfo(jnp.float32).min)


def _attn_body(ctx_ref, sm_ref, sp_ref, q_ref, k_ref, v_ref, kn_ref, vn_ref,
               o_ref):
    b = pl.program_id(0)
    ctx = ctx_ref[b]

    q = q_ref[0, 0]
    k = k_ref[0, 0]
    v = v_ref[0, 0]
    sp = sp_ref[0]

    ov = jnp.zeros((1, _L), jnp.int32)
    for bp in range(_BATCH):
        ov = jnp.where(sp == sm_ref[bp], bp + 1, ov)
    onehot = (jax.lax.broadcasted_iota(jnp.int32, (_BATCH, _L), 0) + 1 == ov
              ).astype(jnp.float32)
    kov = jax.lax.dot(kn_ref[0], onehot,
                      preferred_element_type=jnp.float32)
    vov = jax.lax.dot(vn_ref[0], onehot,
                      preferred_element_type=jnp.float32)
    has = ov > 0
    k = jnp.where(has, kov, k)
    v = jnp.where(has, vov, v)

    s = jax.lax.dot_general(q, k, (((1,), (0,)), ((), ())),
                            preferred_element_type=jnp.float32) * _SCALE
    mask = jax.lax.broadcasted_iota(jnp.int32, (1, _L), 1) < ctx
    s = jnp.where(mask, s, _NEG)
    m = jnp.max(s, axis=1, keepdims=True)
    e = jnp.exp(s - m)
    den = jnp.sum(e, axis=1, keepdims=True)
    o = jax.lax.dot_general(e, v, (((1,), (1,)), ((), ())),
                            preferred_element_type=jnp.float32)
    o_ref[0, 0] = o / den


def _attention(kg, vg, q_r, knT, vnT, slot_pos, slot_map, context_lens,
               interpret=False):
    grid = (_BATCH, _NUM_KV_HEADS)
    return pl.pallas_call(
        _attn_body,
        grid=grid,
        in_specs=[
            pl.BlockSpec(memory_space=pltpu.SMEM),
            pl.BlockSpec(memory_space=pltpu.SMEM),
            pl.BlockSpec((1, 1, _L), lambda b, h: (b, 0, 0)),
            pl.BlockSpec((1, 1, _QPK, _HEAD), lambda b, h: (b, h, 0, 0)),
            pl.BlockSpec((1, 1, _HEAD, _L), lambda b, h: (b, h, 0, 0)),
            pl.BlockSpec((1, 1, _HEAD, _L), lambda b, h: (b, h, 0, 0)),
            pl.BlockSpec((1, _HEAD, _BATCH), lambda b, h: (h, 0, 0)),
            pl.BlockSpec((1, _HEAD, _BATCH), lambda b, h: (h, 0, 0)),
        ],
        out_specs=pl.BlockSpec((1, 1, _QPK, _HEAD), lambda b, h: (b, h, 0, 0)),
        out_shape=jax.ShapeDtypeStruct(
            (_BATCH, _NUM_KV_HEADS, _QPK, _HEAD), jnp.float32),
        interpret=interpret,
    )(context_lens, slot_map, slot_pos, q_r, kg, vg, knT, vnT)


_GROUP = 16
_NUM_GROUPS = _NUM_PAGES // _GROUP
_TASKS = _NUM_GROUPS * _BATCH * _NUM_KV_HEADS
_ROWS = _GROUP * _HEAD


def _gather_body(kc, vc, bt, ctx, kg, vg, btbuf, ctxbuf, idx2, bufa, bufb,
                 gsem, wsem):
    w = jax.lax.axis_index("s") * 2 + jax.lax.axis_index("c")
    pltpu.sync_copy(ctx, ctxbuf)
    ctxv = ctxbuf[...]
    lane = jax.lax.iota(jnp.int32, 16)
    lane128 = lane * _HEAD

    def task(t, _):
        tid = t * 32 + w
        g = tid // (_BATCH * _NUM_KV_HEADS)
        b = (tid // _NUM_KV_HEADS) % _BATCH
        h = tid % _NUM_KV_HEADS
        ctx_b = jnp.max(jnp.where(lane == b, ctxv, 0))

        @pl.when(g * (_GROUP * _PAGE) < ctx_b)
        def _():
            pltpu.sync_copy(bt.at[pl.ds(b * _NUM_PAGES + g * _GROUP, _GROUP)],
                            btbuf)
            vbase = btbuf[...] * (_NUM_KV_HEADS * _HEAD) + h * _HEAD

            def build(d, _):
                plsc.store_scatter(idx2, [lane128 + d], vbase + d)
                return 0

            jax.lax.fori_loop(0, _HEAD, build, 0)
            for src, dst in ((kc, kg), (vc, vg)):
                pltpu.async_copy(src.at[idx2], bufa, gsem).wait()

                def tr(d, _):
                    for p in range(_GROUP):
                        bufb[d, pl.ds(p * _PAGE, _PAGE)] = bufa[p * _HEAD + d]
                    return 0

                jax.lax.fori_loop(0, _HEAD, tr, 0)
                pltpu.async_copy(
                    bufb, dst.at[b, h, :, pl.ds(g * _GROUP * _PAGE,
                                                _GROUP * _PAGE)], wsem).wait()
        return 0

    jax.lax.fori_loop(0, _TASKS // 32, task, 0)


def _sc_gather(kc_rows, vc_rows, bt_flat, ctx):
    out = jax.ShapeDtypeStruct(
        (_BATCH, _NUM_KV_HEADS, _HEAD, _L), jnp.float32)
    fn = pl.kernel(
        _gather_body,
        out_type=(out, out),
        mesh=plsc.VectorSubcoreMesh(core_axis_name="c", subcore_axis_name="s"),
        compiler_params=pltpu.CompilerParams(use_tc_tiling_on_sc=False,
                                             needs_layout_passes=False),
        scratch_types=[
            pltpu.VMEM((_GROUP,), jnp.int32),
            pltpu.VMEM((16,), jnp.int32),
            pltpu.VMEM((_ROWS,), jnp.int32),
            pltpu.VMEM((_ROWS, _PAGE), jnp.float32),
            pltpu.VMEM((_HEAD, _GROUP * _PAGE), jnp.float32),
            pltpu.SemaphoreType.DMA,
            pltpu.SemaphoreType.DMA,
        ],
    )
    return fn(kc_rows, vc_rows, bt_flat, ctx)


def kernel(query, key, value, key_cache, value_cache, slot_mapping,
           block_tables, context_lens):
    batch, q_len, hidden = query.shape
    q_r = query.reshape(_BATCH, _NUM_KV_HEADS, _QPK, _HEAD)
    knT = jnp.transpose(key.reshape(_BATCH, _NUM_KV_HEADS, _HEAD), (1, 2, 0))
    vnT = jnp.transpose(value.reshape(_BATCH, _NUM_KV_HEADS, _HEAD), (1, 2, 0))
    sm_flat = slot_mapping.reshape(-1).astype(jnp.int32)
    slot_pos = (block_tables[:, :, None] * _PAGE +
                jnp.arange(_PAGE, dtype=jnp.int32)[None, None, :]
                ).reshape(_BATCH, 1, _L)
    bt_flat = block_tables.reshape(-1).astype(jnp.int32)

    kc_rows = key_cache.reshape(-1, _PAGE)
    vc_rows = value_cache.reshape(-1, _PAGE)
    kg, vg = _sc_gather(kc_rows, vc_rows, bt_flat,
                        context_lens.astype(jnp.int32))
    out = _attention(kg, vg, q_r, knT, vnT, slot_pos, sm_flat,
                     context_lens.astype(jnp.int32))
    return out.reshape(batch, q_len, hidden)


# speedup vs baseline: 1.5034x; 1.0357x over previous
import functools

import jax
import jax.numpy as jnp
from jax.experimental import pallas as pl
from jax.experimental.pallas import tpu as pltpu
from jax.experimental.pallas import tpu_sc as plsc

_NUM_HEADS = 16
_NUM_KV_HEADS = 4
_QPK = _NUM_HEADS // _NUM_KV_HEADS
_HEAD = 128
_SCALE = 0.08838834764831845
_PAGE = 16
_NUM_PAGES = 128
_L = _PAGE * _NUM_PAGES
_BATCH = 16
_NEG = float(jnp.finfo(jnp.float32).min)


def _attn_body(ctx_ref, sm_ref, sp_ref, q_ref, k_ref, v_ref, kn_ref, vn_ref,
               o_ref):
    b = pl.program_id(0)
    ctx = ctx_ref[b]

    q = q_ref[0, 0]
    k = k_ref[0, 0]
    v = v_ref[0, 0]
    sp = sp_ref[0]

    ov = jnp.zeros((1, _L), jnp.int32)
    for bp in range(_BATCH):
        ov = jnp.where(sp == sm_ref[bp], bp + 1, ov)
    onehot = (jax.lax.broadcasted_iota(jnp.int32, (_BATCH, _L), 0) + 1 == ov
              ).astype(jnp.float32)
    has = ov > 0

    s = jax.lax.dot_general(q, k, (((1,), (0,)), ((), ())),
                            preferred_element_type=jnp.float32)
    s_ov = jax.lax.dot_general(q, kn_ref[0], (((1,), (0,)), ((), ())),
                               preferred_element_type=jnp.float32)
    s_ovl = jax.lax.dot(s_ov, onehot, preferred_element_type=jnp.float32)
    s = jnp.where(has, s_ovl, s) * _SCALE
    mask = jax.lax.broadcasted_iota(jnp.int32, (1, _L), 1) < ctx
    s = jnp.where(mask, s, _NEG)
    m = jnp.max(s, axis=1, keepdims=True)
    e = jnp.exp(s - m)
    den = jnp.sum(e, axis=1, keepdims=True)
    e_stale = jnp.where(has, 0.0, e)
    e_ov = jax.lax.dot_general(e, onehot, (((1,), (1,)), ((), ())),
                               preferred_element_type=jnp.float32)
    o = jax.lax.dot_general(e_stale, v, (((1,), (1,)), ((), ())),
                            preferred_element_type=jnp.float32)
    o += jax.lax.dot_general(e_ov, vn_ref[0], (((1,), (1,)), ((), ())),
                             preferred_element_type=jnp.float32)
    o_ref[0, 0] = o / den


def _attention(kg, vg, q_r, knT, vnT, slot_pos, slot_map, context_lens,
               interpret=False):
    grid = (_BATCH, _NUM_KV_HEADS)
    return pl.pallas_call(
        _attn_body,
        grid=grid,
        in_specs=[
            pl.BlockSpec(memory_space=pltpu.SMEM),
            pl.BlockSpec(memory_space=pltpu.SMEM),
            pl.BlockSpec((1, 1, _L), lambda b, h: (b, 0, 0)),
            pl.BlockSpec((1, 1, _QPK, _HEAD), lambda b, h: (b, h, 0, 0)),
            pl.BlockSpec((1, 1, _HEAD, _L), lambda b, h: (b, h, 0, 0)),
            pl.BlockSpec((1, 1, _HEAD, _L), lambda b, h: (b, h, 0, 0)),
            pl.BlockSpec((1, _HEAD, _BATCH), lambda b, h: (h, 0, 0)),
            pl.BlockSpec((1, _HEAD, _BATCH), lambda b, h: (h, 0, 0)),
        ],
        out_specs=pl.BlockSpec((1, 1, _QPK, _HEAD), lambda b, h: (b, h, 0, 0)),
        out_shape=jax.ShapeDtypeStruct(
            (_BATCH, _NUM_KV_HEADS, _QPK, _HEAD), jnp.float32),
        interpret=interpret,
    )(context_lens, slot_map, slot_pos, q_r, kg, vg, knT, vnT)


_GROUP = 16
_NUM_GROUPS = _NUM_PAGES // _GROUP
_TASKS = _NUM_GROUPS * _BATCH * _NUM_KV_HEADS
_ROWS = _GROUP * _HEAD


def _gather_body(kc, vc, bt, ctx, kg, vg, btbuf, ctxbuf, idx2, bufa, bufb,
                 gsem, wsem):
    w = jax.lax.axis_index("s") * 2 + jax.lax.axis_index("c")
    pltpu.sync_copy(ctx, ctxbuf)
    ctxv = ctxbuf[...]
    lane = jax.lax.iota(jnp.int32, 16)

    def task(t, _):
        tid = t * 32 + w
        g = tid // (_BATCH * _NUM_KV_HEADS)
        b = (tid // _NUM_KV_HEADS) % _BATCH
        h = tid % _NUM_KV_HEADS
        ctx_b = jnp.max(jnp.where(lane == b, ctxv, 0))

        @pl.when(g * (_GROUP * _PAGE) < ctx_b)
        def _():
            pltpu.sync_copy(bt.at[pl.ds(b * _NUM_PAGES + g * _GROUP, _GROUP)],
                            btbuf)
            idx2[...] = btbuf[...] * _NUM_KV_HEADS + h
            for src, dst in ((kc, kg), (vc, vg)):
                pltpu.async_copy(src.at[idx2], bufa, gsem).wait()

                def tr(d, _):
                    for p in range(_GROUP):
                        bufb[d, pl.ds(p * _PAGE, _PAGE)] = (
                            bufa[p, pl.ds(d * _PAGE, _PAGE)])
                    return 0

                jax.lax.fori_loop(0, _HEAD, tr, 0)
                pltpu.async_copy(
                    bufb, dst.at[b, h, :, pl.ds(g * _GROUP * _PAGE,
                                                _GROUP * _PAGE)], wsem).wait()
        return 0

    jax.lax.fori_loop(0, _TASKS // 32, task, 0)


def _sc_gather(kc_rows, vc_rows, bt_flat, ctx):
    out = jax.ShapeDtypeStruct(
        (_BATCH, _NUM_KV_HEADS, _HEAD, _L), jnp.float32)
    fn = pl.kernel(
        _gather_body,
        out_type=(out, out),
        mesh=plsc.VectorSubcoreMesh(core_axis_name="c", subcore_axis_name="s"),
        compiler_params=pltpu.CompilerParams(use_tc_tiling_on_sc=False,
                                             needs_layout_passes=False),
        scratch_types=[
            pltpu.VMEM((_GROUP,), jnp.int32),
            pltpu.VMEM((16,), jnp.int32),
            pltpu.VMEM((_GROUP,), jnp.int32),
            pltpu.VMEM((_GROUP, _HEAD * _PAGE), jnp.float32),
            pltpu.VMEM((_HEAD, _GROUP * _PAGE), jnp.float32),
            pltpu.SemaphoreType.DMA,
            pltpu.SemaphoreType.DMA,
        ],
    )
    return fn(kc_rows, vc_rows, bt_flat, ctx)


def kernel(query, key, value, key_cache, value_cache, slot_mapping,
           block_tables, context_lens):
    batch, q_len, hidden = query.shape
    q_r = query.reshape(_BATCH, _NUM_KV_HEADS, _QPK, _HEAD)
    knT = jnp.transpose(key.reshape(_BATCH, _NUM_KV_HEADS, _HEAD), (1, 2, 0))
    vnT = jnp.transpose(value.reshape(_BATCH, _NUM_KV_HEADS, _HEAD), (1, 2, 0))
    sm_flat = slot_mapping.reshape(-1).astype(jnp.int32)
    slot_pos = (block_tables[:, :, None] * _PAGE +
                jnp.arange(_PAGE, dtype=jnp.int32)[None, None, :]
                ).reshape(_BATCH, 1, _L)
    bt_flat = block_tables.reshape(-1).astype(jnp.int32)

    kc_rows = key_cache.reshape(-1, _HEAD * _PAGE)
    vc_rows = value_cache.reshape(-1, _HEAD * _PAGE)
    kg, vg = _sc_gather(kc_rows, vc_rows, bt_flat,
                        context_lens.astype(jnp.int32))
    out = _attention(kg, vg, q_r, knT, vnT, slot_pos, sm_flat,
                     context_lens.astype(jnp.int32))
    return out.reshape(batch, q_len, hidden)


# speedup vs baseline: 1.5043x; 1.0006x over previous
import functools

import jax
import jax.numpy as jnp
from jax.experimental import pallas as pl
from jax.experimental.pallas import tpu as pltpu
from jax.experimental.pallas import tpu_sc as plsc

_NUM_HEADS = 16
_NUM_KV_HEADS = 4
_QPK = _NUM_HEADS // _NUM_KV_HEADS
_HEAD = 128
_SCALE = 0.08838834764831845
_PAGE = 16
_NUM_BLOCKS = 2048
_NUM_PAGES = 128
_L = _PAGE * _NUM_PAGES
_BATCH = 16
_NEG = float(jnp.finfo(jnp.float32).min)


def _attn_body(ctx_ref, sm_ref, sp_ref, q_ref, k_ref, v_ref, kn_ref, vn_ref,
               o_ref):
    b = pl.program_id(0)
    ctx = ctx_ref[b]

    q = q_ref[0, 0]
    k = k_ref[0, 0]
    v = v_ref[0, 0]
    sp = sp_ref[0]

    ov = jnp.zeros((1, _L), jnp.int32)
    for bp in range(_BATCH):
        ov = jnp.where(sp == sm_ref[bp], bp + 1, ov)
    onehot = (jax.lax.broadcasted_iota(jnp.int32, (_BATCH, _L), 0) + 1 == ov
              ).astype(jnp.float32)
    has = ov > 0

    s = jax.lax.dot_general(q, k, (((1,), (0,)), ((), ())),
                            preferred_element_type=jnp.float32)
    s_ov = jax.lax.dot_general(q, kn_ref[0], (((1,), (0,)), ((), ())),
                               preferred_element_type=jnp.float32)
    s_ovl = jax.lax.dot(s_ov, onehot, preferred_element_type=jnp.float32)
    s = jnp.where(has, s_ovl, s) * _SCALE
    mask = jax.lax.broadcasted_iota(jnp.int32, (1, _L), 1) < ctx
    s = jnp.where(mask, s, _NEG)
    m = jnp.max(s, axis=1, keepdims=True)
    e = jnp.exp(s - m)
    den = jnp.sum(e, axis=1, keepdims=True)
    e_stale = jnp.where(has, 0.0, e)
    e_ov = jax.lax.dot_general(e, onehot, (((1,), (1,)), ((), ())),
                               preferred_element_type=jnp.float32)
    o = jax.lax.dot_general(e_stale, v, (((1,), (1,)), ((), ())),
                            preferred_element_type=jnp.float32)
    o += jax.lax.dot_general(e_ov, vn_ref[0], (((1,), (1,)), ((), ())),
                             preferred_element_type=jnp.float32)
    o_ref[0, 0] = o / den


def _attention(kg, vg, q_r, knT, vnT, slot_pos, slot_map, context_lens,
               interpret=False):
    grid = (_BATCH, _NUM_KV_HEADS)
    return pl.pallas_call(
        _attn_body,
        grid=grid,
        in_specs=[
            pl.BlockSpec(memory_space=pltpu.SMEM),
            pl.BlockSpec(memory_space=pltpu.SMEM),
            pl.BlockSpec((1, 1, _L), lambda b, h: (b, 0, 0)),
            pl.BlockSpec((1, 1, _QPK, _HEAD), lambda b, h: (b, h, 0, 0)),
            pl.BlockSpec((1, 1, _HEAD, _L), lambda b, h: (b, h, 0, 0)),
            pl.BlockSpec((1, 1, _HEAD, _L), lambda b, h: (b, h, 0, 0)),
            pl.BlockSpec((1, _HEAD, _BATCH), lambda b, h: (h, 0, 0)),
            pl.BlockSpec((1, _HEAD, _BATCH), lambda b, h: (h, 0, 0)),
        ],
        out_specs=pl.BlockSpec((1, 1, _QPK, _HEAD), lambda b, h: (b, h, 0, 0)),
        out_shape=jax.ShapeDtypeStruct(
            (_BATCH, _NUM_KV_HEADS, _QPK, _HEAD), jnp.float32),
        interpret=interpret,
    )(context_lens, slot_map, slot_pos, q_r, kg, vg, knT, vnT)


_GROUP = 16
_NUM_GROUPS = _NUM_PAGES // _GROUP
_TASKS = _NUM_GROUPS * _BATCH * _NUM_KV_HEADS
_ROWS = _GROUP * _HEAD


def _gather_body(kc, vc, bt, ctx, kg, vg, btbuf, ctxbuf, idx2, bufa, bufb,
                 gsem, wsem):
    w = jax.lax.axis_index("s") * 2 + jax.lax.axis_index("c")
    pltpu.sync_copy(ctx, ctxbuf)
    ctxv = ctxbuf[...]
    lane = jax.lax.iota(jnp.int32, 16)

    def task(t, _):
        tid = t * 32 + w
        g = tid // (_BATCH * _NUM_KV_HEADS)
        b = (tid // _NUM_KV_HEADS) % _BATCH
        h = tid % _NUM_KV_HEADS
        ctx_b = jnp.max(jnp.where(lane == b, ctxv, 0))

        @pl.when(g * (_GROUP * _PAGE) < ctx_b)
        def _():
            pltpu.sync_copy(bt.at[pl.ds(b * _NUM_PAGES + g * _GROUP, _GROUP)],
                            btbuf)
            idx2[...] = btbuf[...] * _NUM_KV_HEADS + h
            for src, dst in ((kc, kg), (vc, vg)):
                pltpu.async_copy(src.at[idx2], bufa, gsem).wait()

                def tr(d, _):
                    for p in range(_GROUP):
                        bufb[d, pl.ds(p * _PAGE, _PAGE)] = bufa[p, d, :]
                    return 0

                jax.lax.fori_loop(0, _HEAD, tr, 0)
                pltpu.async_copy(
                    bufb, dst.at[b, h, :, pl.ds(g * _GROUP * _PAGE,
                                                _GROUP * _PAGE)], wsem).wait()
        return 0

    jax.lax.fori_loop(0, _TASKS // 32, task, 0)


def _sc_gather(kc_rows, vc_rows, bt_flat, ctx):
    out = jax.ShapeDtypeStruct(
        (_BATCH, _NUM_KV_HEADS, _HEAD, _L), jnp.float32)
    fn = pl.kernel(
        _gather_body,
        out_type=(out, out),
        mesh=plsc.VectorSubcoreMesh(core_axis_name="c", subcore_axis_name="s"),
        compiler_params=pltpu.CompilerParams(use_tc_tiling_on_sc=False,
                                             needs_layout_passes=False),
        scratch_types=[
            pltpu.VMEM((_GROUP,), jnp.int32),
            pltpu.VMEM((16,), jnp.int32),
            pltpu.VMEM((_GROUP,), jnp.int32),
            pltpu.VMEM((_GROUP, _HEAD, _PAGE), jnp.float32),
            pltpu.VMEM((_HEAD, _GROUP * _PAGE), jnp.float32),
            pltpu.SemaphoreType.DMA,
            pltpu.SemaphoreType.DMA,
        ],
    )
    return fn(kc_rows, vc_rows, bt_flat, ctx)


def kernel(query, key, value, key_cache, value_cache, slot_mapping,
           block_tables, context_lens):
    batch, q_len, hidden = query.shape
    q_r = query.reshape(_BATCH, _NUM_KV_HEADS, _QPK, _HEAD)
    knT = jnp.transpose(key.reshape(_BATCH, _NUM_KV_HEADS, _HEAD), (1, 2, 0))
    vnT = jnp.transpose(value.reshape(_BATCH, _NUM_KV_HEADS, _HEAD), (1, 2, 0))
    sm_flat = slot_mapping.reshape(-1).astype(jnp.int32)
    slot_pos = (block_tables[:, :, None] * _PAGE +
                jnp.arange(_PAGE, dtype=jnp.int32)[None, None, :]
                ).reshape(_BATCH, 1, _L)
    bt_flat = block_tables.reshape(-1).astype(jnp.int32)

    kc3 = key_cache.reshape(_NUM_BLOCKS * _NUM_KV_HEADS, _HEAD, _PAGE)
    vc3 = value_cache.reshape(_NUM_BLOCKS * _NUM_KV_HEADS, _HEAD, _PAGE)
    kg, vg = _sc_gather(kc3, vc3, bt_flat, context_lens.astype(jnp.int32))
    out = _attention(kg, vg, q_r, knT, vnT, slot_pos, sm_flat,
                     context_lens.astype(jnp.int32))
    return out.reshape(batch, q_len, hidden)
